# Initial kernel scaffold; baseline (speedup 1.0000x reference)
import jax
import jax.numpy as jnp
from jax.experimental import pallas as pl


def kernel(h, edge_index, allDBGEmb, gPos, W_init, b_init, bn1_scale, bn1_bias, bn2_scale, bn2_bias, W_fc, b_fc, bn3_scale, bn3_bias, W_fc2, b_fc2):
    raise NotImplementedError("write your pallas kernel here")


# speedup vs baseline: 2.1421x; 2.1421x over previous
import functools

import jax
import jax.numpy as jnp
from jax import lax
from jax.experimental import pallas as pl
from jax.experimental.pallas import tpu as pltpu
from jax.experimental.pallas import tpu_sc as plsc

_N = 10000
_E = 160000
_H = 1024
_G = 4096
_W32 = 128
_CW = 128
_NCH = 8

_NW = 32
_EPW = 5120
_NB = 40
_K = 128
_ACC_ROWS = 10112
_STRIP = _ACC_ROWS // 16
_EPS = 1e-5

_HI = lax.Precision.HIGHEST


def _dot(a, b):
    return jax.lax.dot_general(a, b, (((a.ndim - 1,), (0,)), ((), ())),
                               precision=_HI, preferred_element_type=jnp.float32)


def _sc_edge_agg(src3, dst3, tables, width):
    n_t = len(tables)
    mesh = plsc.VectorSubcoreMesh(core_axis_name="c", subcore_axis_name="s")
    zeros_hbm = jnp.zeros((_STRIP, width), jnp.float32)

    @functools.partial(
        pl.kernel,
        mesh=mesh,
        out_type=[jax.ShapeDtypeStruct((2 * _ACC_ROWS, width), jnp.float32)] * n_t,
        scratch_types=[
            pltpu.VMEM((_NB, _K), jnp.int32),
            pltpu.VMEM((_NB, _K), jnp.int32),
            pltpu.VMEM((_K, width), jnp.float32),
            pltpu.VMEM_SHARED((_ACC_ROWS, width), jnp.float32),
            pltpu.SemaphoreType.DMA,
        ],
    )
    def k(src_hbm, dst_hbm, z_hbm, *rest):
        tbls = rest[:n_t]
        outs = rest[n_t:2 * n_t]
        src_v, dst_v, rows_v, acc, sem = rest[2 * n_t:]
        cid = lax.axis_index("c")
        sid = lax.axis_index("s")
        wid = sid * 2 + cid
        pltpu.sync_copy(src_hbm.at[wid], src_v)
        pltpu.sync_copy(dst_hbm.at[wid], dst_v)
        for t in range(n_t):
            pltpu.sync_copy(z_hbm, acc.at[pl.ds(sid * _STRIP, _STRIP)])
            plsc.subcore_barrier()

            def body(j, carry, t=t):
                pltpu.async_copy(tbls[t].at[src_v.at[j]], rows_v, sem).wait()
                pltpu.sync_copy(rows_v, acc.at[dst_v.at[j]], add=True)
                return carry

            lax.fori_loop(0, _NB, body, 0)
            plsc.subcore_barrier()
            pltpu.sync_copy(
                acc.at[pl.ds(sid * _STRIP, _STRIP)],
                outs[t].at[pl.ds(cid * _ACC_ROWS + sid * _STRIP, _STRIP)])
            plsc.subcore_barrier()

    res = k(src3, dst3, zeros_hbm, *tables)
    return list(res) if isinstance(res, (list, tuple)) else [res]


def _sc_gather(table, idx):
    B = idx.shape[0]
    bpw = B // _NW
    half = bpw // 2
    mesh = plsc.VectorSubcoreMesh(core_axis_name="c", subcore_axis_name="s")

    @functools.partial(
        pl.kernel,
        mesh=mesh,
        out_type=jax.ShapeDtypeStruct((B, _H), jnp.float32),
        scratch_types=[
            pltpu.VMEM((bpw,), jnp.int32),
            pltpu.VMEM((half, _H), jnp.float32),
            pltpu.SemaphoreType.DMA,
        ],
    )
    def k(tbl_hbm, idx_hbm, out_hbm, idx_v, rows_v, sem):
        cid = lax.axis_index("c")
        sid = lax.axis_index("s")
        wid = sid * 2 + cid
        base = wid * bpw
        pltpu.sync_copy(idx_hbm.at[pl.ds(base, bpw)], idx_v)
        for b in range(2):
            pltpu.async_copy(tbl_hbm.at[idx_v.at[pl.ds(b * half, half)]],
                             rows_v, sem).wait()
            pltpu.sync_copy(rows_v, out_hbm.at[pl.ds(base + b * half, half)])

    return k(table, idx)


def _l1_body(hp_ref, p1_ref, wb_ref, s1_ref, b1_ref, u_ref, a1_ref, bb1_ref):
    hp = hp_ref[...]
    s = p1_ref[0] + p1_ref[1]
    cnt = s[:, 20:21]
    rcnt = 1.0 / jnp.maximum(cnt, 1.0)
    u = hp + s * jnp.broadcast_to(rcnt, (_N, _W32))
    lane = lax.broadcasted_iota(jnp.int32, (_N, _W32), 1)
    ind = jnp.broadcast_to((cnt > 0).astype(jnp.float32), (_N, _W32))
    u = jnp.where(lane < 20, u, 0.0)
    u = jnp.where(lane == 20, 1.0 + ind, u)
    u_ref[...] = u

    wb = wb_ref[...]
    colsum = jnp.sum(u, axis=0, keepdims=True)
    m2 = jax.lax.dot_general(u, u, (((0,), (0,)), ((), ())),
                             precision=_HI, preferred_element_type=jnp.float32)
    mean1 = _dot(colsum, wb) * (1.0 / _N)
    t1 = _dot(m2, wb)
    meansq = jnp.sum(wb * t1, axis=0, keepdims=True) * (1.0 / _N)
    var1 = meansq - mean1 * mean1
    a1 = s1_ref[...] * jax.lax.rsqrt(var1 + _EPS)
    a1_ref[...] = a1
    bb1_ref[...] = b1_ref[...] - mean1 * a1


def _tc_layer1(h_pad, p1, wb, bn1_s, bn1_b):
    return pl.pallas_call(
        _l1_body,
        out_shape=[
            jax.ShapeDtypeStruct((_N, _W32), jnp.float32),
            jax.ShapeDtypeStruct((1, _H), jnp.float32),
            jax.ShapeDtypeStruct((1, _H), jnp.float32),
        ],
    )(h_pad, p1, wb, bn1_s, bn1_b)


def _zc_body(u_ref, wb_ref, a1_ref, b1_ref, *z_refs):
    y = _dot(u_ref[...], wb_ref[...])
    a1 = a1_ref[...]
    b1 = b1_ref[...]
    for c in range(_NCH):
        sl = slice(c * _CW, (c + 1) * _CW)
        z_refs[c][...] = jnp.maximum(y[:, sl] * a1[:, sl] + b1[:, sl], 0.0)


def _tc_z(u, wb, a1, b1, rows_tile=1000):
    nt = _N // rows_tile
    return pl.pallas_call(
        _zc_body,
        grid=(nt,),
        in_specs=[
            pl.BlockSpec((rows_tile, _W32), lambda i: (i, 0)),
            pl.BlockSpec((_W32, _H), lambda i: (0, 0)),
            pl.BlockSpec((1, _H), lambda i: (0, 0)),
            pl.BlockSpec((1, _H), lambda i: (0, 0)),
        ],
        out_specs=[pl.BlockSpec((rows_tile, _CW), lambda i: (i, 0))] * _NCH,
        out_shape=[jax.ShapeDtypeStruct((_N, _CW), jnp.float32)] * _NCH,
    )(u, wb, a1, b1)


def _h3_chunks(p1_ref, z_refs, p2_refs, rows_tile):
    s = p1_ref[0] + p1_ref[1]
    cnt = s[:, 20:21]
    rcnt = jnp.broadcast_to(1.0 / jnp.maximum(cnt, 1.0), (rows_tile, _CW))
    for c in range(_NCH):
        yield z_refs[c][...] + (p2_refs[c][0] + p2_refs[c][1]) * rcnt


def _stats_body(p1_ref, *refs, rows_tile):
    z_refs = refs[:_NCH]
    p2_refs = refs[_NCH:2 * _NCH]
    sum_ref, sq_ref = refs[2 * _NCH:]

    @pl.when(pl.program_id(0) == 0)
    def _():
        sum_ref[...] = jnp.zeros((_NCH, _CW), jnp.float32)
        sq_ref[...] = jnp.zeros((_NCH, _CW), jnp.float32)

    sums, sqs = [], []
    for h3c in _h3_chunks(p1_ref, z_refs, p2_refs, rows_tile):
        sums.append(jnp.sum(h3c, axis=0, keepdims=True))
        sqs.append(jnp.sum(h3c * h3c, axis=0, keepdims=True))
    sum_ref[...] += jnp.concatenate(sums, axis=0)
    sq_ref[...] += jnp.concatenate(sqs, axis=0)


def _qsum_body(p1_ref, *refs, rows_tile):
    z_refs = refs[:_NCH]
    p2_refs = refs[_NCH:2 * _NCH]
    sum_in, sq_in, s2_ref, b2_ref, q_ref = refs[2 * _NCH:]

    mean2 = sum_in[...] * (1.0 / _N)
    var2 = sq_in[...] * (1.0 / _N) - mean2 * mean2
    a2 = s2_ref[...] * jax.lax.rsqrt(var2 + _EPS)
    b2 = b2_ref[...] - mean2 * a2

    @pl.when(pl.program_id(0) == 0)
    def _():
        q_ref[...] = jnp.zeros((_NCH, _CW), jnp.float32)

    qs = []
    for c, h3c in enumerate(_h3_chunks(p1_ref, z_refs, p2_refs, rows_tile)):
        zc = jnp.maximum(h3c * a2[c:c + 1, :] + b2[c:c + 1, :], 0.0)
        qs.append(jnp.sum(zc, axis=0, keepdims=True))
    q_ref[...] += jnp.concatenate(qs, axis=0)


def _tc_stats_and_qsum(p1, zs, p2s, bn2_s8, bn2_b8, rows_tile=1000):
    nt = _N // rows_tile
    base_specs = (
        [pl.BlockSpec((2, rows_tile, _W32), lambda i: (0, i, 0))]
        + [pl.BlockSpec((rows_tile, _CW), lambda i: (i, 0))] * _NCH
        + [pl.BlockSpec((2, rows_tile, _CW), lambda i: (0, i, 0))] * _NCH
    )
    const8 = pl.BlockSpec((_NCH, _CW), lambda i: (0, 0))
    sumr, sqr = pl.pallas_call(
        functools.partial(_stats_body, rows_tile=rows_tile),
        grid=(nt,),
        in_specs=base_specs,
        out_specs=[const8, const8],
        out_shape=[jax.ShapeDtypeStruct((_NCH, _CW), jnp.float32)] * 2,
    )(p1, *zs, *p2s)
    qsum = pl.pallas_call(
        functools.partial(_qsum_body, rows_tile=rows_tile),
        grid=(nt,),
        in_specs=base_specs + [const8] * 4,
        out_specs=const8,
        out_shape=jax.ShapeDtypeStruct((_NCH, _CW), jnp.float32),
    )(p1, *zs, *p2s, sumr, sqr, bn2_s8, bn2_b8)
    return qsum


def _head_body(g_ref, wfb_ref, wft_ref, q_ref, bfc_ref, s3_ref, b3_ref,
               w2_ref, b2s_ref, out_ref):
    qn = q_ref[...] * (1.0 / _N)
    qv = _dot(qn[0:1, :], wft_ref[0])
    for c in range(1, _NCH):
        qv = qv + _dot(qn[c:c + 1, :], wft_ref[c])
    p = _dot(g_ref[...], wfb_ref[...]) + qv + bfc_ref[...]
    m3 = jnp.sum(p, axis=0, keepdims=True) * (1.0 / _G)
    cen = p - m3
    v3 = jnp.sum(cen * cen, axis=0, keepdims=True) * (1.0 / _G)
    h2 = jnp.maximum(cen * (s3_ref[...] * jax.lax.rsqrt(v3 + _EPS))
                     + b3_ref[...], 0.0)
    logits = jnp.sum(h2 * w2_ref[...], axis=1, keepdims=True) + b2s_ref[...]
    out_ref[...] = 1.0 / (1.0 + jnp.exp(-logits))


def _tc_head(gemb, wfcb, wfct3, qsum, bfc, bn3_s, bn3_b, w2row, b2s):
    return pl.pallas_call(
        _head_body,
        out_shape=jax.ShapeDtypeStruct((_G, 1), jnp.float32),
    )(gemb, wfcb, wfct3, qsum, bfc, bn3_s, bn3_b, w2row, b2s)


def kernel(h, edge_index, allDBGEmb, gPos, W_init, b_init,
           bn1_scale, bn1_bias, bn2_scale, bn2_bias,
           W_fc, b_fc, bn3_scale, bn3_bias, W_fc2, b_fc2):
    src = edge_index[0]
    dst = edge_index[1]
    pad = _NW * _EPW - _E
    src3 = jnp.concatenate([src, jnp.zeros((pad,), jnp.int32)]
                           ).reshape(_NW, _NB, _K)
    dst3 = jnp.concatenate([dst, jnp.full((pad,), _N, jnp.int32)]
                           ).reshape(_NW, _NB, _K)

    h_pad = jnp.concatenate(
        [h, jnp.ones((_N, 1), jnp.float32),
         jnp.zeros((_N, _W32 - 21), jnp.float32)], axis=1)
    wb = jnp.concatenate(
        [W_init, b_init[None, :], jnp.zeros((_W32 - 21, _H), jnp.float32)],
        axis=0)

    (p1_flat,) = _sc_edge_agg(src3, dst3, [h_pad], _W32)
    p1 = jnp.stack([p1_flat[:_N], p1_flat[_ACC_ROWS:_ACC_ROWS + _N]])
    u, a1, b1 = _tc_layer1(h_pad, p1, wb,
                           bn1_scale[None, :], bn1_bias[None, :])
    zs = _tc_z(u, wb, a1, b1)

    p2_flat = _sc_edge_agg(src3, dst3, list(zs), _CW)
    p2s = [jnp.stack([p[:_N], p[_ACC_ROWS:_ACC_ROWS + _N]]) for p in p2_flat]

    qsum = _tc_stats_and_qsum(p1, zs, p2s,
                              bn2_scale.reshape(_NCH, _CW),
                              bn2_bias.reshape(_NCH, _CW))

    gemb = _sc_gather(allDBGEmb, gPos)
    probs = _tc_head(gemb, W_fc[_H:], W_fc[:_H].reshape(_NCH, _CW, _CW),
                     qsum, b_fc[None, :], bn3_scale[None, :],
                     bn3_bias[None, :], W_fc2[:, 0][None, :],
                     b_fc2[None, :])
    return probs.reshape(-1)


# speedup vs baseline: 2.4086x; 1.1244x over previous
import functools

import jax
import jax.numpy as jnp
from jax import lax
from jax.experimental import pallas as pl
from jax.experimental.pallas import tpu as pltpu
from jax.experimental.pallas import tpu_sc as plsc

_N = 10000
_E = 160000
_H = 1024
_G = 4096
_W32 = 128
_CW = 128
_NCH = 8

_NW = 32
_EPW = 5120
_NB = 40
_K = 128
_ACC_ROWS = 10112
_STRIP = _ACC_ROWS // 16
_EPS = 1e-5

_HI = lax.Precision.HIGHEST


def _dot(a, b):
    return jax.lax.dot_general(a, b, (((a.ndim - 1,), (0,)), ((), ())),
                               precision=_HI, preferred_element_type=jnp.float32)


def _sc_edge_agg(src3, dst3, tables, width):
    n_t = len(tables)
    mesh = plsc.VectorSubcoreMesh(core_axis_name="c", subcore_axis_name="s")
    zeros_hbm = jnp.zeros((_STRIP, width), jnp.float32)

    @functools.partial(
        pl.kernel,
        mesh=mesh,
        out_type=[jax.ShapeDtypeStruct((2 * _ACC_ROWS, width), jnp.float32)] * n_t,
        scratch_types=[
            pltpu.VMEM((_NB, _K), jnp.int32),
            pltpu.VMEM((_NB, _K), jnp.int32),
            pltpu.VMEM((_K, width), jnp.float32),
            pltpu.VMEM((_K, width), jnp.float32),
            pltpu.VMEM_SHARED((_ACC_ROWS, width), jnp.float32),
            pltpu.SemaphoreType.DMA,
            pltpu.SemaphoreType.DMA,
            pltpu.SemaphoreType.DMA,
            pltpu.SemaphoreType.DMA,
        ],
    )
    def k(src_hbm, dst_hbm, z_hbm, *rest):
        tbls = rest[:n_t]
        outs = rest[n_t:2 * n_t]
        src_v, dst_v, r0, r1, acc, g0, g1, s0, s1 = rest[2 * n_t:]
        rows = (r0, r1)
        gsem = (g0, g1)
        ssem = (s0, s1)
        cid = lax.axis_index("c")
        sid = lax.axis_index("s")
        wid = sid * 2 + cid
        pltpu.sync_copy(src_hbm.at[wid], src_v)
        pltpu.sync_copy(dst_hbm.at[wid], dst_v)
        for t in range(n_t):
            tbl = tbls[t]
            pltpu.sync_copy(z_hbm, acc.at[pl.ds(sid * _STRIP, _STRIP)])
            plsc.subcore_barrier()

            for p in range(2):
                pltpu.async_copy(tbl.at[src_v.at[p]], rows[p], gsem[p])

            def body(i, carry, tbl=tbl):
                for p in range(2):
                    j = 2 * i + p
                    pltpu.make_async_copy(
                        tbl.at[src_v.at[j]], rows[p], gsem[p]).wait()
                    pltpu.async_copy(rows[p], acc.at[dst_v.at[j]],
                                     ssem[p], add=True)
                    pltpu.make_async_copy(
                        rows[p], acc.at[dst_v.at[j]], ssem[p]).wait()
                    pltpu.async_copy(tbl.at[src_v.at[j + 2]],
                                     rows[p], gsem[p])
                return carry

            lax.fori_loop(0, _NB // 2 - 1, body, 0)
            for p in range(2):
                j = _NB - 2 + p
                pltpu.make_async_copy(
                    tbl.at[src_v.at[j]], rows[p], gsem[p]).wait()
                pltpu.async_copy(rows[p], acc.at[dst_v.at[j]],
                                 ssem[p], add=True)
                pltpu.make_async_copy(
                    rows[p], acc.at[dst_v.at[j]], ssem[p]).wait()
            plsc.subcore_barrier()
            pltpu.sync_copy(
                acc.at[pl.ds(sid * _STRIP, _STRIP)],
                outs[t].at[pl.ds(cid * _ACC_ROWS + sid * _STRIP, _STRIP)])
            plsc.subcore_barrier()

    res = k(src3, dst3, zeros_hbm, *tables)
    return list(res) if isinstance(res, (list, tuple)) else [res]


def _sc_gather(table, idx):
    B = idx.shape[0]
    bpw = B // _NW
    half = bpw // 2
    mesh = plsc.VectorSubcoreMesh(core_axis_name="c", subcore_axis_name="s")

    @functools.partial(
        pl.kernel,
        mesh=mesh,
        out_type=jax.ShapeDtypeStruct((B, _H), jnp.float32),
        scratch_types=[
            pltpu.VMEM((bpw,), jnp.int32),
            pltpu.VMEM((half, _H), jnp.float32),
            pltpu.SemaphoreType.DMA,
        ],
    )
    def k(tbl_hbm, idx_hbm, out_hbm, idx_v, rows_v, sem):
        cid = lax.axis_index("c")
        sid = lax.axis_index("s")
        wid = sid * 2 + cid
        base = wid * bpw
        pltpu.sync_copy(idx_hbm.at[pl.ds(base, bpw)], idx_v)
        for b in range(2):
            pltpu.async_copy(tbl_hbm.at[idx_v.at[pl.ds(b * half, half)]],
                             rows_v, sem).wait()
            pltpu.sync_copy(rows_v, out_hbm.at[pl.ds(base + b * half, half)])

    return k(table, idx)


def _l1_body(hp_ref, p1_ref, wb_ref, s1_ref, b1_ref, u_ref, a1_ref, bb1_ref):
    hp = hp_ref[...]
    s = p1_ref[0] + p1_ref[1]
    cnt = s[:, 20:21]
    rcnt = 1.0 / jnp.maximum(cnt, 1.0)
    u = hp + s * jnp.broadcast_to(rcnt, (_N, _W32))
    lane = lax.broadcasted_iota(jnp.int32, (_N, _W32), 1)
    ind = jnp.broadcast_to((cnt > 0).astype(jnp.float32), (_N, _W32))
    u = jnp.where(lane < 20, u, 0.0)
    u = jnp.where(lane == 20, 1.0 + ind, u)
    u_ref[...] = u

    wb = wb_ref[...]
    colsum = jnp.sum(u, axis=0, keepdims=True)
    m2 = jax.lax.dot_general(u, u, (((0,), (0,)), ((), ())),
                             precision=_HI, preferred_element_type=jnp.float32)
    mean1 = _dot(colsum, wb) * (1.0 / _N)
    t1 = _dot(m2, wb)
    meansq = jnp.sum(wb * t1, axis=0, keepdims=True) * (1.0 / _N)
    var1 = meansq - mean1 * mean1
    a1 = s1_ref[...] * jax.lax.rsqrt(var1 + _EPS)
    a1_ref[...] = a1
    bb1_ref[...] = b1_ref[...] - mean1 * a1


def _tc_layer1(h_pad, p1, wb, bn1_s, bn1_b):
    return pl.pallas_call(
        _l1_body,
        out_shape=[
            jax.ShapeDtypeStruct((_N, _W32), jnp.float32),
            jax.ShapeDtypeStruct((1, _H), jnp.float32),
            jax.ShapeDtypeStruct((1, _H), jnp.float32),
        ],
    )(h_pad, p1, wb, bn1_s, bn1_b)


def _zc_body(u_ref, wb_ref, a1_ref, b1_ref, *z_refs):
    y = _dot(u_ref[...], wb_ref[...])
    a1 = a1_ref[...]
    b1 = b1_ref[...]
    for c in range(_NCH):
        sl = slice(c * _CW, (c + 1) * _CW)
        z_refs[c][...] = jnp.maximum(y[:, sl] * a1[:, sl] + b1[:, sl], 0.0)


def _tc_z(u, wb, a1, b1, rows_tile=1000):
    nt = _N // rows_tile
    return pl.pallas_call(
        _zc_body,
        grid=(nt,),
        in_specs=[
            pl.BlockSpec((rows_tile, _W32), lambda i: (i, 0)),
            pl.BlockSpec((_W32, _H), lambda i: (0, 0)),
            pl.BlockSpec((1, _H), lambda i: (0, 0)),
            pl.BlockSpec((1, _H), lambda i: (0, 0)),
        ],
        out_specs=[pl.BlockSpec((rows_tile, _CW), lambda i: (i, 0))] * _NCH,
        out_shape=[jax.ShapeDtypeStruct((_N, _CW), jnp.float32)] * _NCH,
    )(u, wb, a1, b1)


def _h3_chunks(p1_ref, z_refs, p2_refs, rows_tile):
    s = p1_ref[0] + p1_ref[1]
    cnt = s[:, 20:21]
    rcnt = jnp.broadcast_to(1.0 / jnp.maximum(cnt, 1.0), (rows_tile, _CW))
    for c in range(_NCH):
        yield z_refs[c][...] + (p2_refs[c][0] + p2_refs[c][1]) * rcnt


def _stats_body(p1_ref, *refs, rows_tile):
    z_refs = refs[:_NCH]
    p2_refs = refs[_NCH:2 * _NCH]
    sum_ref, sq_ref = refs[2 * _NCH:]

    @pl.when(pl.program_id(0) == 0)
    def _():
        sum_ref[...] = jnp.zeros((_NCH, _CW), jnp.float32)
        sq_ref[...] = jnp.zeros((_NCH, _CW), jnp.float32)

    sums, sqs = [], []
    for h3c in _h3_chunks(p1_ref, z_refs, p2_refs, rows_tile):
        sums.append(jnp.sum(h3c, axis=0, keepdims=True))
        sqs.append(jnp.sum(h3c * h3c, axis=0, keepdims=True))
    sum_ref[...] += jnp.concatenate(sums, axis=0)
    sq_ref[...] += jnp.concatenate(sqs, axis=0)


def _qsum_body(p1_ref, *refs, rows_tile):
    z_refs = refs[:_NCH]
    p2_refs = refs[_NCH:2 * _NCH]
    sum_in, sq_in, s2_ref, b2_ref, q_ref = refs[2 * _NCH:]

    mean2 = sum_in[...] * (1.0 / _N)
    var2 = sq_in[...] * (1.0 / _N) - mean2 * mean2
    a2 = s2_ref[...] * jax.lax.rsqrt(var2 + _EPS)
    b2 = b2_ref[...] - mean2 * a2

    @pl.when(pl.program_id(0) == 0)
    def _():
        q_ref[...] = jnp.zeros((_NCH, _CW), jnp.float32)

    qs = []
    for c, h3c in enumerate(_h3_chunks(p1_ref, z_refs, p2_refs, rows_tile)):
        zc = jnp.maximum(h3c * a2[c:c + 1, :] + b2[c:c + 1, :], 0.0)
        qs.append(jnp.sum(zc, axis=0, keepdims=True))
    q_ref[...] += jnp.concatenate(qs, axis=0)


def _tc_stats_and_qsum(p1, zs, p2s, bn2_s8, bn2_b8, rows_tile=1000):
    nt = _N // rows_tile
    base_specs = (
        [pl.BlockSpec((2, rows_tile, _W32), lambda i: (0, i, 0))]
        + [pl.BlockSpec((rows_tile, _CW), lambda i: (i, 0))] * _NCH
        + [pl.BlockSpec((2, rows_tile, _CW), lambda i: (0, i, 0))] * _NCH
    )
    const8 = pl.BlockSpec((_NCH, _CW), lambda i: (0, 0))
    sumr, sqr = pl.pallas_call(
        functools.partial(_stats_body, rows_tile=rows_tile),
        grid=(nt,),
        in_specs=base_specs,
        out_specs=[const8, const8],
        out_shape=[jax.ShapeDtypeStruct((_NCH, _CW), jnp.float32)] * 2,
    )(p1, *zs, *p2s)
    qsum = pl.pallas_call(
        functools.partial(_qsum_body, rows_tile=rows_tile),
        grid=(nt,),
        in_specs=base_specs + [const8] * 4,
        out_specs=const8,
        out_shape=jax.ShapeDtypeStruct((_NCH, _CW), jnp.float32),
    )(p1, *zs, *p2s, sumr, sqr, bn2_s8, bn2_b8)
    return qsum


def _head_body(g_ref, wfb_ref, wft_ref, q_ref, bfc_ref, s3_ref, b3_ref,
               w2_ref, b2s_ref, out_ref):
    qn = q_ref[...] * (1.0 / _N)
    qv = _dot(qn[0:1, :], wft_ref[0])
    for c in range(1, _NCH):
        qv = qv + _dot(qn[c:c + 1, :], wft_ref[c])
    p = _dot(g_ref[...], wfb_ref[...]) + qv + bfc_ref[...]
    m3 = jnp.sum(p, axis=0, keepdims=True) * (1.0 / _G)
    cen = p - m3
    v3 = jnp.sum(cen * cen, axis=0, keepdims=True) * (1.0 / _G)
    h2 = jnp.maximum(cen * (s3_ref[...] * jax.lax.rsqrt(v3 + _EPS))
                     + b3_ref[...], 0.0)
    logits = jnp.sum(h2 * w2_ref[...], axis=1, keepdims=True) + b2s_ref[...]
    out_ref[...] = 1.0 / (1.0 + jnp.exp(-logits))


def _tc_head(gemb, wfcb, wfct3, qsum, bfc, bn3_s, bn3_b, w2row, b2s):
    return pl.pallas_call(
        _head_body,
        out_shape=jax.ShapeDtypeStruct((_G, 1), jnp.float32),
    )(gemb, wfcb, wfct3, qsum, bfc, bn3_s, bn3_b, w2row, b2s)


def kernel(h, edge_index, allDBGEmb, gPos, W_init, b_init,
           bn1_scale, bn1_bias, bn2_scale, bn2_bias,
           W_fc, b_fc, bn3_scale, bn3_bias, W_fc2, b_fc2):
    src = edge_index[0]
    dst = edge_index[1]
    pad = _NW * _EPW - _E
    src3 = jnp.concatenate([src, jnp.zeros((pad,), jnp.int32)]
                           ).reshape(_NW, _NB, _K)
    dst3 = jnp.concatenate([dst, jnp.full((pad,), _N, jnp.int32)]
                           ).reshape(_NW, _NB, _K)

    h_pad = jnp.concatenate(
        [h, jnp.ones((_N, 1), jnp.float32),
         jnp.zeros((_N, _W32 - 21), jnp.float32)], axis=1)
    wb = jnp.concatenate(
        [W_init, b_init[None, :], jnp.zeros((_W32 - 21, _H), jnp.float32)],
        axis=0)

    (p1_flat,) = _sc_edge_agg(src3, dst3, [h_pad], _W32)
    p1 = jnp.stack([p1_flat[:_N], p1_flat[_ACC_ROWS:_ACC_ROWS + _N]])
    u, a1, b1 = _tc_layer1(h_pad, p1, wb,
                           bn1_scale[None, :], bn1_bias[None, :])
    zs = _tc_z(u, wb, a1, b1)

    p2_flat = _sc_edge_agg(src3, dst3, list(zs), _CW)
    p2s = [jnp.stack([p[:_N], p[_ACC_ROWS:_ACC_ROWS + _N]]) for p in p2_flat]

    qsum = _tc_stats_and_qsum(p1, zs, p2s,
                              bn2_scale.reshape(_NCH, _CW),
                              bn2_bias.reshape(_NCH, _CW))

    gemb = _sc_gather(allDBGEmb, gPos)
    probs = _tc_head(gemb, W_fc[_H:], W_fc[:_H].reshape(_NCH, _CW, _CW),
                     qsum, b_fc[None, :], bn3_scale[None, :],
                     bn3_bias[None, :], W_fc2[:, 0][None, :],
                     b_fc2[None, :])
    return probs.reshape(-1)


# speedup vs baseline: 2.5276x; 1.0494x over previous
import functools

import jax
import jax.numpy as jnp
from jax import lax
from jax.experimental import pallas as pl
from jax.experimental.pallas import tpu as pltpu
from jax.experimental.pallas import tpu_sc as plsc

_N = 10000
_E = 160000
_H = 1024
_G = 4096
_W32 = 128
_CW = 128
_NCH = 8

_NW = 32
_K = 64
_NBUF = 2
_NB0 = 120
_NB1 = 40
_TOTB = 16 * (_NB0 + _NB1)
_TOTB_PAD = _TOTB + _NB0 - _NB1
_ACC_ROWS = 10112
_STRIP = _ACC_ROWS // 16
_EPS = 1e-5

_HI = lax.Precision.HIGHEST


def _dot(a, b):
    return jax.lax.dot_general(a, b, (((a.ndim - 1,), (0,)), ((), ())),
                               precision=_HI, preferred_element_type=jnp.float32)


def _sc_edge_agg(src3, dst3, tables, width):
    n_t = len(tables)
    mesh = plsc.VectorSubcoreMesh(core_axis_name="c", subcore_axis_name="s")
    zeros_hbm = jnp.zeros((_STRIP, width), jnp.float32)

    @functools.partial(
        pl.kernel,
        mesh=mesh,
        out_type=[jax.ShapeDtypeStruct((2 * _ACC_ROWS, width), jnp.float32)] * n_t,
        scratch_types=[
            pltpu.VMEM((_NB0, _K), jnp.int32),
            pltpu.VMEM((_NB0, _K), jnp.int32),
        ]
        + [pltpu.VMEM((_K, width), jnp.float32)] * _NBUF
        + [pltpu.VMEM_SHARED((_ACC_ROWS, width), jnp.float32)]
        + [pltpu.SemaphoreType.DMA] * (2 * _NBUF),
    )
    def k(src_hbm, dst_hbm, z_hbm, *rest):
        tbls = rest[:n_t]
        outs = rest[n_t:2 * n_t]
        rest = rest[2 * n_t:]
        src_v, dst_v = rest[0], rest[1]
        rows = rest[2:2 + _NBUF]
        acc = rest[2 + _NBUF]
        gsem = rest[3 + _NBUF:3 + 2 * _NBUF]
        ssem = rest[3 + 2 * _NBUF:3 + 3 * _NBUF]
        cid = lax.axis_index("c")
        sid = lax.axis_index("s")
        nb = jnp.where(cid == 0, _NB0, _NB1)
        base = jnp.where(cid == 0, sid * _NB0, 16 * _NB0 + sid * _NB1)
        pltpu.sync_copy(src_hbm.at[pl.ds(base, _NB0)], src_v)
        pltpu.sync_copy(dst_hbm.at[pl.ds(base, _NB0)], dst_v)
        for t in range(n_t):
            tbl = tbls[t]
            pltpu.sync_copy(z_hbm, acc.at[pl.ds(sid * _STRIP, _STRIP)])
            plsc.subcore_barrier()

            for p in range(_NBUF):
                pltpu.async_copy(tbl.at[src_v.at[p]], rows[p], gsem[p])

            def body(i, carry, tbl=tbl):
                for p in range(_NBUF):
                    j = _NBUF * i + p
                    pltpu.make_async_copy(
                        tbl.at[src_v.at[j]], rows[p], gsem[p]).wait()
                    pltpu.async_copy(rows[p], acc.at[dst_v.at[j]],
                                     ssem[p], add=True)
                    pltpu.make_async_copy(
                        rows[p], acc.at[dst_v.at[j]], ssem[p]).wait()
                    pltpu.async_copy(tbl.at[src_v.at[j + _NBUF]],
                                     rows[p], gsem[p])
                return carry

            lax.fori_loop(0, nb // _NBUF - 1, body, 0)
            for p in range(_NBUF):
                j = nb - _NBUF + p
                pltpu.make_async_copy(
                    tbl.at[src_v.at[j]], rows[p], gsem[p]).wait()
                pltpu.async_copy(rows[p], acc.at[dst_v.at[j]],
                                 ssem[p], add=True)
                pltpu.make_async_copy(
                    rows[p], acc.at[dst_v.at[j]], ssem[p]).wait()
            plsc.subcore_barrier()
            pltpu.sync_copy(
                acc.at[pl.ds(sid * _STRIP, _STRIP)],
                outs[t].at[pl.ds(cid * _ACC_ROWS + sid * _STRIP, _STRIP)])
            plsc.subcore_barrier()

    res = k(src3, dst3, zeros_hbm, *tables)
    return list(res) if isinstance(res, (list, tuple)) else [res]


def _sc_gather(table, idx):
    B = idx.shape[0]
    bpw = B // _NW
    half = bpw // 2
    mesh = plsc.VectorSubcoreMesh(core_axis_name="c", subcore_axis_name="s")

    @functools.partial(
        pl.kernel,
        mesh=mesh,
        out_type=jax.ShapeDtypeStruct((B, _H), jnp.float32),
        scratch_types=[
            pltpu.VMEM((bpw,), jnp.int32),
            pltpu.VMEM((half, _H), jnp.float32),
            pltpu.SemaphoreType.DMA,
        ],
    )
    def k(tbl_hbm, idx_hbm, out_hbm, idx_v, rows_v, sem):
        cid = lax.axis_index("c")
        sid = lax.axis_index("s")
        wid = sid * 2 + cid
        base = wid * bpw
        pltpu.sync_copy(idx_hbm.at[pl.ds(base, bpw)], idx_v)
        for b in range(2):
            pltpu.async_copy(tbl_hbm.at[idx_v.at[pl.ds(b * half, half)]],
                             rows_v, sem).wait()
            pltpu.sync_copy(rows_v, out_hbm.at[pl.ds(base + b * half, half)])

    return k(table, idx)


def _l1_body(hp_ref, p1_ref, wb_ref, s1_ref, b1_ref, u_ref, a1_ref, bb1_ref):
    hp = hp_ref[...]
    s = p1_ref[0] + p1_ref[1]
    cnt = s[:, 20:21]
    rcnt = 1.0 / jnp.maximum(cnt, 1.0)
    u = hp + s * jnp.broadcast_to(rcnt, (_N, _W32))
    lane = lax.broadcasted_iota(jnp.int32, (_N, _W32), 1)
    ind = jnp.broadcast_to((cnt > 0).astype(jnp.float32), (_N, _W32))
    u = jnp.where(lane < 20, u, 0.0)
    u = jnp.where(lane == 20, 1.0 + ind, u)
    u_ref[...] = u

    wb = wb_ref[...]
    colsum = jnp.sum(u, axis=0, keepdims=True)
    m2 = jax.lax.dot_general(u, u, (((0,), (0,)), ((), ())),
                             precision=_HI, preferred_element_type=jnp.float32)
    mean1 = _dot(colsum, wb) * (1.0 / _N)
    t1 = _dot(m2, wb)
    meansq = jnp.sum(wb * t1, axis=0, keepdims=True) * (1.0 / _N)
    var1 = meansq - mean1 * mean1
    a1 = s1_ref[...] * jax.lax.rsqrt(var1 + _EPS)
    a1_ref[...] = a1
    bb1_ref[...] = b1_ref[...] - mean1 * a1


def _tc_layer1(h_pad, p1, wb, bn1_s, bn1_b):
    return pl.pallas_call(
        _l1_body,
        out_shape=[
            jax.ShapeDtypeStruct((_N, _W32), jnp.float32),
            jax.ShapeDtypeStruct((1, _H), jnp.float32),
            jax.ShapeDtypeStruct((1, _H), jnp.float32),
        ],
    )(h_pad, p1, wb, bn1_s, bn1_b)


def _zc_body(u_ref, wb_ref, a1_ref, b1_ref, *z_refs):
    y = _dot(u_ref[...], wb_ref[...])
    a1 = a1_ref[...]
    b1 = b1_ref[...]
    for c in range(_NCH):
        sl = slice(c * _CW, (c + 1) * _CW)
        z_refs[c][...] = jnp.maximum(y[:, sl] * a1[:, sl] + b1[:, sl], 0.0)


def _tc_z(u, wb, a1, b1, rows_tile=1000):
    nt = _N // rows_tile
    return pl.pallas_call(
        _zc_body,
        grid=(nt,),
        in_specs=[
            pl.BlockSpec((rows_tile, _W32), lambda i: (i, 0)),
            pl.BlockSpec((_W32, _H), lambda i: (0, 0)),
            pl.BlockSpec((1, _H), lambda i: (0, 0)),
            pl.BlockSpec((1, _H), lambda i: (0, 0)),
        ],
        out_specs=[pl.BlockSpec((rows_tile, _CW), lambda i: (i, 0))] * _NCH,
        out_shape=[jax.ShapeDtypeStruct((_N, _CW), jnp.float32)] * _NCH,
    )(u, wb, a1, b1)


def _h3_chunks(p1_ref, z_refs, p2_refs, rows_tile):
    s = p1_ref[0] + p1_ref[1]
    cnt = s[:, 20:21]
    rcnt = jnp.broadcast_to(1.0 / jnp.maximum(cnt, 1.0), (rows_tile, _CW))
    for c in range(_NCH):
        yield z_refs[c][...] + (p2_refs[c][0] + p2_refs[c][1]) * rcnt


def _stats_body(p1_ref, *refs, rows_tile):
    z_refs = refs[:_NCH]
    p2_refs = refs[_NCH:2 * _NCH]
    sum_ref, sq_ref = refs[2 * _NCH:]

    @pl.when(pl.program_id(0) == 0)
    def _():
        sum_ref[...] = jnp.zeros((_NCH, _CW), jnp.float32)
        sq_ref[...] = jnp.zeros((_NCH, _CW), jnp.float32)

    sums, sqs = [], []
    for h3c in _h3_chunks(p1_ref, z_refs, p2_refs, rows_tile):
        sums.append(jnp.sum(h3c, axis=0, keepdims=True))
        sqs.append(jnp.sum(h3c * h3c, axis=0, keepdims=True))
    sum_ref[...] += jnp.concatenate(sums, axis=0)
    sq_ref[...] += jnp.concatenate(sqs, axis=0)


def _qsum_body(p1_ref, *refs, rows_tile):
    z_refs = refs[:_NCH]
    p2_refs = refs[_NCH:2 * _NCH]
    sum_in, sq_in, s2_ref, b2_ref, q_ref = refs[2 * _NCH:]

    mean2 = sum_in[...] * (1.0 / _N)
    var2 = sq_in[...] * (1.0 / _N) - mean2 * mean2
    a2 = s2_ref[...] * jax.lax.rsqrt(var2 + _EPS)
    b2 = b2_ref[...] - mean2 * a2

    @pl.when(pl.program_id(0) == 0)
    def _():
        q_ref[...] = jnp.zeros((_NCH, _CW), jnp.float32)

    qs = []
    for c, h3c in enumerate(_h3_chunks(p1_ref, z_refs, p2_refs, rows_tile)):
        zc = jnp.maximum(h3c * a2[c:c + 1, :] + b2[c:c + 1, :], 0.0)
        qs.append(jnp.sum(zc, axis=0, keepdims=True))
    q_ref[...] += jnp.concatenate(qs, axis=0)


def _tc_stats_and_qsum(p1, zs, p2s, bn2_s8, bn2_b8, rows_tile=1000):
    nt = _N // rows_tile
    base_specs = (
        [pl.BlockSpec((2, rows_tile, _W32), lambda i: (0, i, 0))]
        + [pl.BlockSpec((rows_tile, _CW), lambda i: (i, 0))] * _NCH
        + [pl.BlockSpec((2, rows_tile, _CW), lambda i: (0, i, 0))] * _NCH
    )
    const8 = pl.BlockSpec((_NCH, _CW), lambda i: (0, 0))
    sumr, sqr = pl.pallas_call(
        functools.partial(_stats_body, rows_tile=rows_tile),
        grid=(nt,),
        in_specs=base_specs,
        out_specs=[const8, const8],
        out_shape=[jax.ShapeDtypeStruct((_NCH, _CW), jnp.float32)] * 2,
    )(p1, *zs, *p2s)
    qsum = pl.pallas_call(
        functools.partial(_qsum_body, rows_tile=rows_tile),
        grid=(nt,),
        in_specs=base_specs + [const8] * 4,
        out_specs=const8,
        out_shape=jax.ShapeDtypeStruct((_NCH, _CW), jnp.float32),
    )(p1, *zs, *p2s, sumr, sqr, bn2_s8, bn2_b8)
    return qsum


def _head_body(g_ref, wfb_ref, wft_ref, q_ref, bfc_ref, s3_ref, b3_ref,
               w2_ref, b2s_ref, out_ref):
    qn = q_ref[...] * (1.0 / _N)
    qv = _dot(qn[0:1, :], wft_ref[0])
    for c in range(1, _NCH):
        qv = qv + _dot(qn[c:c + 1, :], wft_ref[c])
    p = _dot(g_ref[...], wfb_ref[...]) + qv + bfc_ref[...]
    m3 = jnp.sum(p, axis=0, keepdims=True) * (1.0 / _G)
    cen = p - m3
    v3 = jnp.sum(cen * cen, axis=0, keepdims=True) * (1.0 / _G)
    h2 = jnp.maximum(cen * (s3_ref[...] * jax.lax.rsqrt(v3 + _EPS))
                     + b3_ref[...], 0.0)
    logits = jnp.sum(h2 * w2_ref[...], axis=1, keepdims=True) + b2s_ref[...]
    out_ref[...] = 1.0 / (1.0 + jnp.exp(-logits))


def _tc_head(gemb, wfcb, wfct3, qsum, bfc, bn3_s, bn3_b, w2row, b2s):
    return pl.pallas_call(
        _head_body,
        out_shape=jax.ShapeDtypeStruct((_G, 1), jnp.float32),
    )(gemb, wfcb, wfct3, qsum, bfc, bn3_s, bn3_b, w2row, b2s)


def kernel(h, edge_index, allDBGEmb, gPos, W_init, b_init,
           bn1_scale, bn1_bias, bn2_scale, bn2_bias,
           W_fc, b_fc, bn3_scale, bn3_bias, W_fc2, b_fc2):
    src = edge_index[0]
    dst = edge_index[1]
    pad = _TOTB_PAD * _K - _E
    src3 = jnp.concatenate([src, jnp.zeros((pad,), jnp.int32)]
                           ).reshape(_TOTB_PAD, _K)
    dst3 = jnp.concatenate([dst, jnp.full((pad,), _N, jnp.int32)]
                           ).reshape(_TOTB_PAD, _K)

    h_pad = jnp.concatenate(
        [h, jnp.ones((_N, 1), jnp.float32),
         jnp.zeros((_N, _W32 - 21), jnp.float32)], axis=1)
    wb = jnp.concatenate(
        [W_init, b_init[None, :], jnp.zeros((_W32 - 21, _H), jnp.float32)],
        axis=0)

    (p1_flat,) = _sc_edge_agg(src3, dst3, [h_pad], _W32)
    p1 = jnp.stack([p1_flat[:_N], p1_flat[_ACC_ROWS:_ACC_ROWS + _N]])
    u, a1, b1 = _tc_layer1(h_pad, p1, wb,
                           bn1_scale[None, :], bn1_bias[None, :])
    zs = _tc_z(u, wb, a1, b1)

    p2_flat = _sc_edge_agg(src3, dst3, list(zs), _CW)
    p2s = [jnp.stack([p[:_N], p[_ACC_ROWS:_ACC_ROWS + _N]]) for p in p2_flat]

    qsum = _tc_stats_and_qsum(p1, zs, p2s,
                              bn2_scale.reshape(_NCH, _CW),
                              bn2_bias.reshape(_NCH, _CW))

    gemb = _sc_gather(allDBGEmb, gPos)
    probs = _tc_head(gemb, W_fc[_H:], W_fc[:_H].reshape(_NCH, _CW, _CW),
                     qsum, b_fc[None, :], bn3_scale[None, :],
                     bn3_bias[None, :], W_fc2[:, 0][None, :],
                     b_fc2[None, :])
    return probs.reshape(-1)
